# Initial kernel scaffold; baseline (speedup 1.0000x reference)
import jax
import jax.numpy as jnp
from jax.experimental import pallas as pl


def kernel(x, edge_index, edge_attr, W_se, b_se, W1, att_src1, att_dst1, W_e1, att_e1, b1, W2, att_src2, att_dst2, W_e2, att_e2, b2):
    raise NotImplementedError("write your pallas kernel here")


# speedup vs baseline: 25.1589x; 25.1589x over previous
import functools

import jax
import jax.numpy as jnp
from jax import lax
from jax.experimental import pallas as pl
from jax.experimental.pallas import tpu as pltpu
from jax.experimental.pallas import tpu_sc as plsc

f32 = jnp.float32
i32 = jnp.int32

N = 10000
E = 320000
D_IN = 128
HID = 32
HEADS = 8
EDGE_DIM = HID // 4

ETOT = N + E
NC, NS = 2, 16
EPB = 128
NCHUNK = 162
EPS = NCHUNK * EPB
EPAD = NS * EPS
TOTC = NS * NCHUNK
NP = 10240
NPS = NP // NS
ROWS1 = 144
ROWS2 = 32

_mesh = plsc.VectorSubcoreMesh(
    core_axis_name="c", subcore_axis_name="s", num_cores=NC, num_subcores=NS)


_EB = 32000


def _k1a_body(eattrT, wseT, bseT, m1T, m2T, eal1T, eal2T, easum, acc):
    i = pl.program_id(0)
    ea = jnp.maximum(
        jnp.dot(wseT[...], eattrT[...], preferred_element_type=f32)
        + bseT[...], 0.0)
    eal1T[...] = jnp.dot(m1T[...], ea, preferred_element_type=f32)
    eal2T[...] = jnp.dot(m2T[...], ea, preferred_element_type=f32)

    @pl.when(i == 0)
    def _():
        acc[...] = jnp.zeros_like(acc)

    acc[...] += jnp.sum(ea, axis=1, keepdims=True)
    easum[...] = acc[...]


def _k1a(edge_attrT, w_seT, b_seT, m1T, m2T):
    grid = E // _EB
    return pl.pallas_call(
        _k1a_body,
        grid=(grid,),
        in_specs=[
            pl.BlockSpec((2, _EB), lambda i: (0, i)),
            pl.BlockSpec((EDGE_DIM, 2), lambda i: (0, 0)),
            pl.BlockSpec((EDGE_DIM, 1), lambda i: (0, 0)),
            pl.BlockSpec((EDGE_DIM, EDGE_DIM), lambda i: (0, 0)),
            pl.BlockSpec((1, EDGE_DIM), lambda i: (0, 0)),
        ],
        out_specs=[
            pl.BlockSpec((EDGE_DIM, _EB), lambda i: (0, i)),
            pl.BlockSpec((1, _EB), lambda i: (0, i)),
            pl.BlockSpec((EDGE_DIM, 1), lambda i: (0, 0)),
        ],
        out_shape=[
            jax.ShapeDtypeStruct((EDGE_DIM, E), f32),
            jax.ShapeDtypeStruct((1, E), f32),
            jax.ShapeDtypeStruct((EDGE_DIM, 1), f32),
        ],
        scratch_shapes=[pltpu.VMEM((EDGE_DIM, 1), f32)],
    )(edge_attrT, w_seT, b_seT, m1T, m2T)


_NB = 1000


def _k1b_body(x, w1, ssrc, sdst, htab, asrc, adst):
    hb = jnp.dot(x[...], w1[...], preferred_element_type=f32)
    ones = jnp.ones((_NB, 4), f32)
    zeros = jnp.zeros((_NB, 12), f32)
    htab[0] = jnp.concatenate([hb[:, 0:128], ones, zeros], axis=1)
    htab[1] = jnp.concatenate([hb[:, 128:256], ones, zeros], axis=1)
    asv = jnp.dot(hb, ssrc[...], preferred_element_type=f32)
    adv = jnp.dot(hb, sdst[...], preferred_element_type=f32)
    asrc[0] = asv[:, 0:4]
    asrc[1] = asv[:, 4:8]
    adst[0] = adv[:, 0:4]
    adst[1] = adv[:, 4:8]


def _k1b(x, w1, ssrc, sdst):
    grid = N // _NB
    return pl.pallas_call(
        _k1b_body,
        grid=(grid,),
        in_specs=[
            pl.BlockSpec((_NB, D_IN), lambda i: (i, 0)),
            pl.BlockSpec((D_IN, HEADS * HID), lambda i: (0, 0)),
            pl.BlockSpec((HEADS * HID, HEADS), lambda i: (0, 0)),
            pl.BlockSpec((HEADS * HID, HEADS), lambda i: (0, 0)),
        ],
        out_specs=[
            pl.BlockSpec((2, _NB, ROWS1), lambda i: (0, i, 0)),
            pl.BlockSpec((2, _NB, 4), lambda i: (0, i, 0)),
            pl.BlockSpec((2, _NB, 4), lambda i: (0, i, 0)),
        ],
        out_shape=[
            jax.ShapeDtypeStruct((2, N, ROWS1), f32),
            jax.ShapeDtypeStruct((2, N, 4), f32),
            jax.ShapeDtypeStruct((2, N, 4), f32),
        ],
    )(x, w1, ssrc, sdst)


def _k2_body(src_h, dst_h, eal_h, asrc_h, adst_h, htab_h, out_h,
             src_v, dst_v, dsto_v, asv_v, adv_v, eal_v, z_v, rows_v,
             accS, sem):
    c = lax.axis_index("c")
    s = lax.axis_index("s")
    cN = c * N
    lane = lax.iota(i32, 16)
    e_of = lane >> 2
    h_of = lane & 3

    def zrow(g, _):
        e = g // 9
        k = g - e * 9
        rows_v[e, pl.ds(k * 16, 16)] = jnp.zeros((16,), f32)
        return 0
    lax.fori_loop(0, EPB * 9, zrow, 0)
    for k in range(NPS // EPB):
        pltpu.sync_copy(rows_v, accS.at[pl.ds(s * NPS + k * EPB, EPB)])
    plsc.subcore_barrier()

    base = s * EPS

    def chunk(i, _):
        off = base + i * EPB
        q = s * NCHUNK + i
        pltpu.sync_copy(src_h.at[pl.ds(off, EPB)], src_v)
        pltpu.sync_copy(dst_h.at[pl.ds(off, EPB)], dst_v)
        pltpu.sync_copy(eal_h.at[pl.ds((c * TOTC + q) * 4, 4)], eal_v)

        def og(g, _):
            sl = pl.ds(g * 16, 16)
            src_v[sl] = src_v[sl] + cN
            dsto_v[sl] = dst_v[sl] + c * NP
            return 0
        lax.fori_loop(0, EPB // 16, og, 0)

        ca = pltpu.async_copy(asrc_h.at[src_v], asv_v, sem)
        cb = pltpu.async_copy(adst_h.at[dsto_v], adv_v, sem)
        cc = pltpu.async_copy(htab_h.at[src_v], rows_v, sem)
        ca.wait()
        cb.wait()
        cc.wait()

        def zg(g, _):
            i0 = g * 4 + e_of
            sl = pl.ds(g * 16, 16)
            a_s = plsc.load_gather(asv_v, [i0, h_of])
            a_d = plsc.load_gather(adv_v, [i0, h_of])
            ev = plsc.load_gather(eal_v, [h_of, i0])
            t = a_s + a_d + ev
            t = jnp.where(t > 0.0, t, t * 0.2)
            z_v[sl] = jnp.exp(t)
            return 0
        lax.fori_loop(0, EPB // 4, zg, 0)

        def se(e, _):
            e16 = jnp.broadcast_to(e * 4, (16,)).astype(i32)
            for k in range(4):
                m = plsc.load_gather(z_v, [e16 + k])
                sl0 = pl.ds((2 * k) * 16, 16)
                sl1 = pl.ds((2 * k + 1) * 16, 16)
                rows_v[e, sl0] = rows_v[e, sl0] * m
                rows_v[e, sl1] = rows_v[e, sl1] * m
            md = plsc.load_gather(z_v, [e16 + h_of])
            sl = pl.ds(128, 16)
            rows_v[e, sl] = rows_v[e, sl] * md
            return 0
        lax.fori_loop(0, EPB, se, 0)

        pltpu.sync_copy(rows_v, accS.at[dst_v], add=True)
        return 0
    lax.fori_loop(0, NCHUNK, chunk, 0)

    plsc.subcore_barrier()
    for k in range(NPS // EPB):
        r0 = s * NPS + k * EPB
        pltpu.sync_copy(accS.at[pl.ds(r0, EPB)],
                        out_h.at[pl.ds(c * NP + r0, EPB)])


_k2 = functools.partial(
    pl.kernel,
    out_type=jax.ShapeDtypeStruct((2 * NP, ROWS1), f32),
    mesh=_mesh,
    compiler_params=pltpu.CompilerParams(needs_layout_passes=False, use_tc_tiling_on_sc=False),
    scratch_types=[
        pltpu.VMEM((EPB,), i32),
        pltpu.VMEM((EPB,), i32),
        pltpu.VMEM((EPB,), i32),
        pltpu.VMEM((EPB, 4), f32),
        pltpu.VMEM((EPB, 4), f32),
        pltpu.VMEM((4, EPB), f32),
        pltpu.VMEM((EPB * 4,), f32),
        pltpu.VMEM((EPB, ROWS1), f32),
        pltpu.VMEM_SHARED((NP, ROWS1), f32),
        pltpu.SemaphoreType.DMA,
    ],
)(_k2_body)


def _k3_body(acc, w2, r4, b1, as2, ad2, h2x, a2s, a2d):
    a0 = acc[0]
    a1 = acc[1]
    den0 = jnp.dot(a0[:, 128:132], r4[...], preferred_element_type=f32)
    den1 = jnp.dot(a1[:, 128:132], r4[...], preferred_element_type=f32)
    h1 = jnp.concatenate(
        [a0[:, 0:128] / (den0 + 1e-16), a1[:, 0:128] / (den1 + 1e-16)],
        axis=1) + b1[...]
    h1 = jnp.maximum(h1, 0.0)
    h2 = jnp.dot(h1, w2[...], preferred_element_type=f32)
    ones = jnp.ones((_NB, 1), f32)
    zeros = jnp.zeros((_NB, 15), f32)
    h2x[0] = jnp.concatenate([h2[:, 0:16], ones, zeros], axis=1)
    h2x[1] = jnp.concatenate([h2[:, 16:32], ones, zeros], axis=1)
    a2s[...] = jnp.dot(h2, as2[...], preferred_element_type=f32)
    a2d[...] = jnp.dot(h2, ad2[...], preferred_element_type=f32)


def _k3(acc1, w2, r4, b1, as2, ad2):
    grid = N // _NB
    return pl.pallas_call(
        _k3_body,
        grid=(grid,),
        in_specs=[
            pl.BlockSpec((2, _NB, ROWS1), lambda i: (0, i, 0)),
            pl.BlockSpec((HEADS * HID, HID), lambda i: (0, 0)),
            pl.BlockSpec((4, 128), lambda i: (0, 0)),
            pl.BlockSpec((1, HEADS * HID), lambda i: (0, 0)),
            pl.BlockSpec((HID, 1), lambda i: (0, 0)),
            pl.BlockSpec((HID, 1), lambda i: (0, 0)),
        ],
        out_specs=[
            pl.BlockSpec((2, _NB, ROWS2), lambda i: (0, i, 0)),
            pl.BlockSpec((_NB, 1), lambda i: (i, 0)),
            pl.BlockSpec((_NB, 1), lambda i: (i, 0)),
        ],
        out_shape=[
            jax.ShapeDtypeStruct((2, N, ROWS2), f32),
            jax.ShapeDtypeStruct((N, 1), f32),
            jax.ShapeDtypeStruct((N, 1), f32),
        ],
    )(acc1, w2, r4, b1, as2, ad2)


def _k4_body(src_h, dst_h, eal_h, asrc_h, adst_h, htab_h, out_h,
             asrc_v, adst_v, src_v, dst_v, eal_v, z_v, rows_v, accS, sem):
    c = lax.axis_index("c")
    s = lax.axis_index("s")
    cN = c * N

    pltpu.sync_copy(asrc_h, asrc_v)
    pltpu.sync_copy(adst_h, adst_v)

    def zrow(g, _):
        e = g // 2
        k = g - e * 2
        rows_v[e, pl.ds(k * 16, 16)] = jnp.zeros((16,), f32)
        return 0
    lax.fori_loop(0, EPB * 2, zrow, 0)
    for k in range(NPS // EPB):
        pltpu.sync_copy(rows_v, accS.at[pl.ds(s * NPS + k * EPB, EPB)])
    plsc.subcore_barrier()

    base = s * EPS

    def chunk(i, _):
        off = base + i * EPB
        pltpu.sync_copy(src_h.at[pl.ds(off, EPB)], src_v)
        pltpu.sync_copy(dst_h.at[pl.ds(off, EPB)], dst_v)
        pltpu.sync_copy(eal_h.at[pl.ds(off, EPB)], eal_v)

        def zg(g, _):
            sl = pl.ds(g * 16, 16)
            s16 = src_v[sl]
            d16 = dst_v[sl]
            a_s = plsc.load_gather(asrc_v, [s16])
            a_d = plsc.load_gather(adst_v, [d16])
            t = a_s + a_d + eal_v[sl]
            t = jnp.where(t > 0.0, t, t * 0.2)
            z_v[sl] = jnp.exp(t)
            src_v[sl] = s16 + cN
            return 0
        lax.fori_loop(0, EPB // 16, zg, 0)

        pltpu.async_copy(htab_h.at[src_v], rows_v, sem).wait()

        def se(e, _):
            m = plsc.load_gather(z_v, [jnp.broadcast_to(e, (16,)).astype(i32)])
            sl0 = pl.ds(0, 16)
            sl1 = pl.ds(16, 16)
            rows_v[e, sl0] = rows_v[e, sl0] * m
            rows_v[e, sl1] = rows_v[e, sl1] * m
            return 0
        lax.fori_loop(0, EPB, se, 0)

        pltpu.sync_copy(rows_v, accS.at[dst_v], add=True)
        return 0
    lax.fori_loop(0, NCHUNK, chunk, 0)

    plsc.subcore_barrier()
    for k in range(NPS // EPB):
        r0 = s * NPS + k * EPB
        pltpu.sync_copy(accS.at[pl.ds(r0, EPB)],
                        out_h.at[pl.ds(c * NP + r0, EPB)])


_k4 = functools.partial(
    pl.kernel,
    out_type=jax.ShapeDtypeStruct((2 * NP, ROWS2), f32),
    mesh=_mesh,
    compiler_params=pltpu.CompilerParams(needs_layout_passes=False, use_tc_tiling_on_sc=False),
    scratch_types=[
        pltpu.VMEM((N,), f32),
        pltpu.VMEM((NP,), f32),
        pltpu.VMEM((EPB,), i32),
        pltpu.VMEM((EPB,), i32),
        pltpu.VMEM((EPB,), f32),
        pltpu.VMEM((EPB,), f32),
        pltpu.VMEM((EPB, ROWS2), f32),
        pltpu.VMEM_SHARED((NP, ROWS2), f32),
        pltpu.SemaphoreType.DMA,
    ],
)(_k4_body)


def _k5_body(acc2, b2, out):
    a0 = acc2[0]
    a1 = acc2[1]
    d0 = a0[:, 16:17]
    d1 = a1[:, 16:17]
    out[...] = jnp.concatenate(
        [a0[:, 0:16] / (d0 + 1e-16), a1[:, 0:16] / (d1 + 1e-16)],
        axis=1) + b2[...]


def _k5(acc2, b2):
    grid = N // _NB
    return pl.pallas_call(
        _k5_body,
        grid=(grid,),
        in_specs=[
            pl.BlockSpec((2, _NB, ROWS2), lambda i: (0, i, 0)),
            pl.BlockSpec((1, HID), lambda i: (0, 0)),
        ],
        out_specs=pl.BlockSpec((_NB, HID), lambda i: (i, 0)),
        out_shape=jax.ShapeDtypeStruct((N, HID), f32),
    )(acc2, b2)


def kernel(x, edge_index, edge_attr, W_se, b_se, W1, att_src1, att_dst1,
           W_e1, att_e1, b1, W2, att_src2, att_dst2, W_e2, att_e2, b2):
    m1 = (W_e1.reshape(EDGE_DIM, HEADS, HID) * att_e1[None]).sum(-1)
    m2 = (W_e2.reshape(EDGE_DIM, 1, HID) * att_e2[None]).sum(-1)
    eye_h = jnp.eye(HEADS, dtype=f32)
    ssrc = (eye_h[:, None, :] * att_src1[:, :, None]).reshape(HEADS * HID, HEADS)
    sdst = (eye_h[:, None, :] * att_dst1[:, :, None]).reshape(HEADS * HID, HEADS)
    eye4 = jnp.eye(4, dtype=f32)
    r4 = jnp.broadcast_to(eye4[:, :, None], (4, 4, HID)).reshape(4, 128)

    eal1T, eal2T, easum = _k1a(edge_attr.T, W_se.T,
                               b_se.reshape(EDGE_DIM, 1), m1.T, m2.T)
    ea_mean = easum[:, 0] / E
    loop1 = ea_mean @ m1
    loop2 = ea_mean @ m2

    htab, asrc, adst = _k1b(x, W1, ssrc, sdst)

    pad = EPAD - ETOT
    loop_ids = jnp.arange(N, dtype=i32)
    src_ext = jnp.concatenate(
        [edge_index[0], loop_ids, jnp.zeros((pad,), i32)])
    dst_ext = jnp.concatenate(
        [edge_index[1], loop_ids, jnp.full((pad,), N, i32)])
    eal1_ext = jnp.concatenate(
        [eal1T.reshape(2, 4, E),
         jnp.broadcast_to(loop1.reshape(2, 4, 1), (2, 4, N)),
         jnp.zeros((2, 4, pad), f32)], axis=2)
    eal1_r = (eal1_ext.reshape(2, 4, TOTC, EPB)
              .transpose(0, 2, 1, 3).reshape(2 * TOTC * 4, EPB))
    eal2_ext = jnp.concatenate(
        [eal2T[0], jnp.full((N,), loop2[0], f32), jnp.zeros((pad,), f32)])

    asrc_f = asrc.reshape(2 * N, 4)
    adst_p = jnp.concatenate(
        [adst, jnp.zeros((2, NP - N, 4), f32)], axis=1).reshape(2 * NP, 4)

    acc1 = _k2(src_ext, dst_ext, eal1_r, asrc_f, adst_p,
               htab.reshape(2 * N, ROWS1))

    h2x, a2s, a2d = _k3(acc1.reshape(2, NP, ROWS1), W2, r4,
                        b1.reshape(1, HEADS * HID),
                        att_src2.reshape(HID, 1), att_dst2.reshape(HID, 1))

    a2d_p = jnp.concatenate([a2d[:, 0], jnp.zeros((NP - N,), f32)], axis=0)

    acc2 = _k4(src_ext, dst_ext, eal2_ext, a2s[:, 0], a2d_p,
               h2x.reshape(2 * N, ROWS2))

    return _k5(acc2.reshape(2, NP, ROWS2), b2.reshape(1, HID))
